# Initial kernel scaffold; baseline (speedup 1.0000x reference)
import jax
import jax.numpy as jnp
from jax.experimental import pallas as pl


def kernel(x, edge_index, edge_weight, W1, b1, W2, b2):
    raise NotImplementedError("write your pallas kernel here")


# speedup vs baseline: 9.4625x; 9.4625x over previous
import functools

import jax
import jax.numpy as jnp
from jax import lax
from jax.experimental import pallas as pl
from jax.experimental.pallas import tpu as pltpu
from jax.experimental.pallas import tpu_sc as plsc

NC = 2
NS = 16
NW = NC * NS
L = 16

K = 128


def _pad_to(n, m):
    return ((n + m - 1) // m) * m


def _make_deg_kernel(NP, EPT, CH):
    NPT = NP // NS

    @functools.partial(
        pl.kernel,
        out_type=jax.ShapeDtypeStruct((NC, NP), jnp.float32),
        mesh=plsc.VectorSubcoreMesh(core_axis_name="c", subcore_axis_name="s"),
        scratch_types=[
            pltpu.VMEM((K,), jnp.int32),
            pltpu.VMEM((K,), jnp.float32),
            pltpu.VMEM((NPT,), jnp.float32),
            pltpu.VMEM_SHARED((NP,), jnp.float32),
        ],
        compiler_params=pltpu.CompilerParams(
            needs_layout_passes=False, use_tc_tiling_on_sc=False),
    )
    def deg_kernel(dst_hbm, ew_hbm, out_hbm, idx_v, ew_v, buf_v, deg_sh):
        c = lax.axis_index("c")
        s = lax.axis_index("s")
        wid = c * NS + s

        @pl.loop(0, NPT // L)
        def _zero(i):
            buf_v[pl.ds(i * L, L)] = jnp.zeros((L,), jnp.float32)

        pltpu.sync_copy(buf_v, deg_sh.at[pl.ds(s * NPT, NPT)])
        plsc.subcore_barrier()

        @pl.loop(0, CH)
        def _accum(g):
            base = wid * EPT + g * K
            pltpu.sync_copy(dst_hbm.at[pl.ds(base, K)], idx_v)
            pltpu.sync_copy(ew_hbm.at[pl.ds(base, K)], ew_v)
            pltpu.sync_copy(ew_v, deg_sh.at[idx_v], add=True)

        plsc.subcore_barrier()
        pltpu.sync_copy(deg_sh.at[pl.ds(s * NPT, NPT)], buf_v)
        pltpu.sync_copy(buf_v, out_hbm.at[c, pl.ds(s * NPT, NPT)])

    return deg_kernel


def _make_agg_kernel(NP, EPT, CH, F):
    RPT = NP // NS

    @functools.partial(
        pl.kernel,
        out_type=jax.ShapeDtypeStruct((NC, NP, F), jnp.float32),
        mesh=plsc.VectorSubcoreMesh(core_axis_name="c", subcore_axis_name="s"),
        scratch_types=[
            pltpu.VMEM((K,), jnp.int32),
            pltpu.VMEM((K,), jnp.int32),
            pltpu.VMEM((K,), jnp.float32),
            pltpu.VMEM((K, F), jnp.float32),
            pltpu.VMEM_SHARED((NP, F), jnp.float32),
            pltpu.SemaphoreType.DMA,
        ],
        compiler_params=pltpu.CompilerParams(
            needs_layout_passes=False, use_tc_tiling_on_sc=False),
    )
    def agg_kernel(src_hbm, dst_hbm, ew_hbm, h_hbm, out_hbm,
                   src_v, dst_v, ew_v, rows_v, agg_sh, sem):
        c = lax.axis_index("c")
        s = lax.axis_index("s")
        wid = c * NS + s

        @pl.loop(0, K)
        def _zero(r):
            for j in range(F // L):
                rows_v[r, pl.ds(j * L, L)] = jnp.zeros((L,), jnp.float32)

        @pl.loop(0, RPT // K)
        def _init(m):
            pltpu.sync_copy(rows_v, agg_sh.at[pl.ds(s * RPT + m * K, K)])

        plsc.subcore_barrier()

        @pl.loop(0, CH)
        def _edges(g):
            base = wid * EPT + g * K
            pltpu.sync_copy(src_hbm.at[pl.ds(base, K)], src_v)
            pltpu.sync_copy(dst_hbm.at[pl.ds(base, K)], dst_v)
            pltpu.sync_copy(ew_hbm.at[pl.ds(base, K)], ew_v)
            pltpu.async_copy(h_hbm.at[src_v], rows_v, sem).wait()

            @pl.loop(0, K)
            def _scale(r):
                w = plsc.load_gather(ew_v, [jnp.broadcast_to(r, (L,))])
                for j in range(F // L):
                    rows_v[r, pl.ds(j * L, L)] = rows_v[r, pl.ds(j * L, L)] * w

            pltpu.sync_copy(rows_v, agg_sh.at[dst_v], add=True)

        plsc.subcore_barrier()

        @pl.loop(0, RPT // K)
        def _dump(m):
            off = s * RPT + m * K
            pltpu.sync_copy(agg_sh.at[pl.ds(off, K)], rows_v)
            pltpu.sync_copy(rows_v, out_hbm.at[c, pl.ds(off, K)])

    return agg_kernel


def _tc1_body(deg_ref, x_ref, w_ref, dis_ref, h_ref):
    deg = deg_ref[:, 0:1] + deg_ref[:, 1:2]
    safe = jnp.where(deg > 0, deg, 1.0)
    dis = jnp.where(deg > 0, lax.rsqrt(safe), 0.0)
    dis_ref[...] = dis
    h = jnp.dot(x_ref[...], w_ref[...], preferred_element_type=jnp.float32,
                precision=lax.Precision.HIGHEST)
    h_ref[...] = h * dis


def _tc2_body(p_ref, dis_ref, b_ref, w_ref, h_ref):
    dis = dis_ref[...]
    z = (p_ref[0] + p_ref[1]) * dis + b_ref[...]
    z = jnp.maximum(z, 0.0)
    h = jnp.dot(z, w_ref[...], preferred_element_type=jnp.float32,
                precision=lax.Precision.HIGHEST)
    h_ref[...] = h * dis


def _tc3_body(p_ref, dis_ref, b_ref, logits_ref, soft_ref):
    logits = (p_ref[0] + p_ref[1]) * dis_ref[...] + b_ref[...]
    logits_ref[...] = logits
    m = jnp.max(logits, axis=1, keepdims=True)
    e = jnp.exp(logits - m)
    soft_ref[...] = e / jnp.sum(e, axis=1, keepdims=True)


def kernel(x, edge_index, edge_weight, W1, b1, W2, b2):
    N, D = x.shape
    H = W1.shape[1]
    C = W2.shape[1]
    E = edge_index.shape[1]

    NP = _pad_to(N, NS * L * NC)
    EPT = _pad_to(-(-E // NW), K)
    E_pad = EPT * NW
    CH = EPT // K

    src = jnp.pad(edge_index[0].astype(jnp.int32), (0, E_pad - E))
    dst = jnp.pad(edge_index[1].astype(jnp.int32), (0, E_pad - E))
    ew = jnp.pad(edge_weight, (0, E_pad - E))
    x_pad = jnp.pad(x, ((0, NP - N), (0, 0)))

    deg2 = _make_deg_kernel(NP, EPT, CH)(dst, ew)
    deg2t = deg2.T

    dis, h1p = pl.pallas_call(
        _tc1_body,
        out_shape=(jax.ShapeDtypeStruct((NP, 1), jnp.float32),
                   jax.ShapeDtypeStruct((NP, H), jnp.float32)),
    )(deg2t, x_pad, W1)

    agg1 = _make_agg_kernel(NP, EPT, CH, H)(src, dst, ew, h1p)

    h2p = pl.pallas_call(
        _tc2_body,
        out_shape=jax.ShapeDtypeStruct((NP, C), jnp.float32),
    )(agg1, dis, b1.reshape(1, H), W2)

    agg2 = _make_agg_kernel(NP, EPT, CH, C)(src, dst, ew, h2p)

    logits, soft = pl.pallas_call(
        _tc3_body,
        out_shape=(jax.ShapeDtypeStruct((NP, C), jnp.float32),
                   jax.ShapeDtypeStruct((NP, C), jnp.float32)),
    )(agg2, dis, b2.reshape(1, C))

    return logits[:N], soft[:N]


# speedup vs baseline: 15.6071x; 1.6494x over previous
import functools

import jax
import jax.numpy as jnp
from jax import lax
from jax.experimental import pallas as pl
from jax.experimental.pallas import tpu as pltpu
from jax.experimental.pallas import tpu_sc as plsc

NC = 2
NS = 16
NW = NC * NS
L = 16

K = 128

_GDN = lax.GatherDimensionNumbers(
    offset_dims=(), collapsed_slice_dims=(0,), start_index_map=(0,))

_SC_PARAMS = pltpu.CompilerParams(
    needs_layout_passes=False, use_tc_tiling_on_sc=False)

_MESH = dict(core_axis_name="c", subcore_axis_name="s")


def _pad_to(n, m):
    return ((n + m - 1) // m) * m


def _lane_bcast(vec, r):
    idx = jnp.full((L, 1), r, dtype=jnp.int32)
    return lax.gather(vec, idx, _GDN, slice_sizes=(1,),
                      mode=lax.GatherScatterMode.PROMISE_IN_BOUNDS)


def _make_deg_kernel(NP, NSUP):
    G = 8
    NPT = NP // NS
    NP2 = NSUP // 2

    @functools.partial(
        pl.kernel,
        out_type=jax.ShapeDtypeStruct((NC, NP), jnp.float32),
        mesh=plsc.VectorSubcoreMesh(**_MESH),
        scratch_types=[
            pltpu.VMEM((G, K), jnp.int32),
            pltpu.VMEM((G, K), jnp.float32),
            pltpu.VMEM((G, K), jnp.int32),
            pltpu.VMEM((G, K), jnp.float32),
            pltpu.VMEM_SHARED((NP,), jnp.float32),
            pltpu.SemaphoreType.DMA,
            pltpu.SemaphoreType.DMA,
        ],
        compiler_params=_SC_PARAMS,
    )
    def deg_kernel(dst_hbm, ew_hbm, out_hbm,
                   dst0, ew0, dst1, ew1, deg_sh, sem0, sem1):
        c = lax.axis_index("c")
        s = lax.axis_index("s")
        wid = c * NS + s
        bufs = ((dst0, ew0, sem0), (dst1, ew1, sem1))

        for j in range(NPT // K):
            @pl.loop(0, K // L)
            def _zero(q):
                ew0[j, pl.ds(q * L, L)] = jnp.zeros((L,), jnp.float32)
        for j in range(NPT // K):
            pltpu.sync_copy(ew0.at[j], deg_sh.at[pl.ds(s * NPT + j * K, K)])
        plsc.subcore_barrier()

        def load(t, b):
            dstb, ewb, _ = bufs[b]
            row0 = (wid * NSUP + t) * G
            pltpu.sync_copy(dst_hbm.at[pl.ds(row0, G)], dstb)
            pltpu.sync_copy(ew_hbm.at[pl.ds(row0, G)], ewb)

        def proc(b):
            dstb, ewb, sem = bufs[b]
            for j in range(G):
                pltpu.async_copy(ewb.at[j], deg_sh.at[dstb.at[j]], sem,
                                 add=True)

        def drain(b):
            dstb, ewb, sem = bufs[b]
            for j in range(G):
                pltpu.make_async_copy(ewb.at[j], deg_sh.at[dstb.at[j]],
                                      sem).wait()

        load(0, 0)

        @pl.loop(0, NP2)
        def _pipe(p):
            t0 = 2 * p

            @pl.when(p > 0)
            def _():
                drain(1)

            load(t0 + 1, 1)
            proc(0)
            drain(0)

            @pl.when(p < NP2 - 1)
            def _():
                load(t0 + 2, 0)

            proc(1)

        drain(1)
        plsc.subcore_barrier()
        for j in range(NPT // K):
            off = s * NPT + j * K
            pltpu.sync_copy(deg_sh.at[pl.ds(off, K)], ew0.at[j])
            pltpu.sync_copy(ew0.at[j], out_hbm.at[c, pl.ds(off, K)])

    return deg_kernel


def _make_agg_kernel(NP, NSUP, G, F):
    SUP = G * K
    RPT = NP // NS
    NP2 = NSUP // 2

    @functools.partial(
        pl.kernel,
        out_type=jax.ShapeDtypeStruct((NC, NP, F), jnp.float32),
        mesh=plsc.VectorSubcoreMesh(**_MESH),
        scratch_types=[
            pltpu.VMEM((G, K), jnp.int32),
            pltpu.VMEM((G, K), jnp.int32),
            pltpu.VMEM((G, K), jnp.float32),
            pltpu.VMEM((SUP, F), jnp.float32),
            pltpu.VMEM((G, K), jnp.int32),
            pltpu.VMEM((G, K), jnp.int32),
            pltpu.VMEM((G, K), jnp.float32),
            pltpu.VMEM((SUP, F), jnp.float32),
            pltpu.VMEM_SHARED((NP, F), jnp.float32),
            pltpu.SemaphoreType.DMA,
            pltpu.SemaphoreType.DMA,
            pltpu.SemaphoreType.DMA,
            pltpu.SemaphoreType.DMA,
        ],
        compiler_params=_SC_PARAMS,
    )
    def agg_kernel(src_hbm, dst_hbm, ew_hbm, h_hbm, out_hbm,
                   src0, dst0, ew0, rows0, src1, dst1, ew1, rows1,
                   agg_sh, gsem0, gsem1, ssem0, ssem1):
        c = lax.axis_index("c")
        s = lax.axis_index("s")
        wid = c * NS + s
        bufs = ((src0, dst0, ew0, rows0, gsem0, ssem0),
                (src1, dst1, ew1, rows1, gsem1, ssem1))

        @pl.loop(0, K)
        def _zero(r):
            for f in range(F // L):
                rows0[r, pl.ds(f * L, L)] = jnp.zeros((L,), jnp.float32)
        for m in range(RPT // K):
            pltpu.sync_copy(rows0.at[pl.ds(0, K)],
                            agg_sh.at[pl.ds(s * RPT + m * K, K)])
        plsc.subcore_barrier()

        def load(t, b):
            srcb, dstb, ewb, rowsb, gsem, _ = bufs[b]
            row0 = (wid * NSUP + t) * G
            pltpu.sync_copy(src_hbm.at[pl.ds(row0, G)], srcb)
            pltpu.sync_copy(dst_hbm.at[pl.ds(row0, G)], dstb)
            pltpu.sync_copy(ew_hbm.at[pl.ds(row0, G)], ewb)
            for j in range(G):
                pltpu.async_copy(h_hbm.at[srcb.at[j]],
                                 rowsb.at[pl.ds(j * K, K)], gsem)

        def proc(b):
            srcb, dstb, ewb, rowsb, gsem, ssem = bufs[b]
            for j in range(G):
                pltpu.make_async_copy(h_hbm.at[srcb.at[j]],
                                      rowsb.at[pl.ds(j * K, K)], gsem).wait()
            for j in range(G):
                @pl.loop(0, K // L)
                def _scale(q):
                    w16 = ewb[j, pl.ds(q * L, L)]
                    for r in range(L):
                        bc = _lane_bcast(w16, r)
                        row = j * K + q * L + r
                        for f in range(F // L):
                            rows_slice = rowsb[row, pl.ds(f * L, L)]
                            rowsb[row, pl.ds(f * L, L)] = rows_slice * bc

                pltpu.async_copy(rowsb.at[pl.ds(j * K, K)],
                                 agg_sh.at[dstb.at[j]], ssem, add=True)

        def drain_scat(b):
            srcb, dstb, ewb, rowsb, gsem, ssem = bufs[b]
            for j in range(G):
                pltpu.make_async_copy(rowsb.at[pl.ds(j * K, K)],
                                      agg_sh.at[dstb.at[j]], ssem).wait()

        load(0, 0)

        @pl.loop(0, NP2)
        def _pipe(p):
            t0 = 2 * p

            @pl.when(p > 0)
            def _():
                drain_scat(1)

            load(t0 + 1, 1)
            proc(0)
            drain_scat(0)

            @pl.when(p < NP2 - 1)
            def _():
                load(t0 + 2, 0)

            proc(1)

        drain_scat(1)
        plsc.subcore_barrier()
        for m in range(RPT // K):
            off = s * RPT + m * K
            pltpu.sync_copy(agg_sh.at[pl.ds(off, K)], rows0.at[pl.ds(0, K)])
            pltpu.sync_copy(rows0.at[pl.ds(0, K)], out_hbm.at[c, pl.ds(off, K)])

    return agg_kernel


def _tc1_body(deg_ref, x_ref, w_ref, dis_ref, h_ref):
    deg = deg_ref[:, 0:1] + deg_ref[:, 1:2]
    safe = jnp.where(deg > 0, deg, 1.0)
    dis = jnp.where(deg > 0, lax.rsqrt(safe), 0.0)
    dis_ref[...] = dis
    h = jnp.dot(x_ref[...], w_ref[...], preferred_element_type=jnp.float32,
                precision=lax.Precision.HIGHEST)
    h_ref[...] = h * dis


def _tc2_body(p_ref, dis_ref, b_ref, w_ref, h_ref):
    dis = dis_ref[...]
    z = (p_ref[0] + p_ref[1]) * dis + b_ref[...]
    z = jnp.maximum(z, 0.0)
    h = jnp.dot(z, w_ref[...], preferred_element_type=jnp.float32,
                precision=lax.Precision.HIGHEST)
    h_ref[...] = h * dis


def _tc3_body(p_ref, dis_ref, b_ref, logits_ref, soft_ref):
    logits = (p_ref[0] + p_ref[1]) * dis_ref[...] + b_ref[...]
    logits_ref[...] = logits
    m = jnp.max(logits, axis=1, keepdims=True)
    e = jnp.exp(logits - m)
    soft_ref[...] = e / jnp.sum(e, axis=1, keepdims=True)


def kernel(x, edge_index, edge_weight, W1, b1, W2, b2):
    N, D = x.shape
    H = W1.shape[1]
    C = W2.shape[1]
    E = edge_index.shape[1]

    NP = _pad_to(N, NS * L * NC)
    SUPER = 1024
    EPT = _pad_to(-(-E // NW), 2 * SUPER)
    E_pad = EPT * NW

    src = jnp.pad(edge_index[0].astype(jnp.int32), (0, E_pad - E))
    dst = jnp.pad(edge_index[1].astype(jnp.int32), (0, E_pad - E))
    ew = jnp.pad(edge_weight, (0, E_pad - E))
    src2 = src.reshape(E_pad // K, K)
    dst2 = dst.reshape(E_pad // K, K)
    ew2 = ew.reshape(E_pad // K, K)
    x_pad = jnp.pad(x, ((0, NP - N), (0, 0)))

    deg2 = _make_deg_kernel(NP, EPT // SUPER * (SUPER // (8 * K)))(dst2, ew2)
    deg2t = deg2.T

    dis, h1p = pl.pallas_call(
        _tc1_body,
        out_shape=(jax.ShapeDtypeStruct((NP, 1), jnp.float32),
                   jax.ShapeDtypeStruct((NP, H), jnp.float32)),
    )(deg2t, x_pad, W1)

    agg1 = _make_agg_kernel(NP, EPT // (4 * K), 4, H)(src2, dst2, ew2, h1p)

    h2p = pl.pallas_call(
        _tc2_body,
        out_shape=jax.ShapeDtypeStruct((NP, C), jnp.float32),
    )(agg1, dis, b1.reshape(1, H), W2)

    agg2 = _make_agg_kernel(NP, EPT // (8 * K), 8, C)(src2, dst2, ew2, h2p)

    logits, soft = pl.pallas_call(
        _tc3_body,
        out_shape=(jax.ShapeDtypeStruct((NP, C), jnp.float32),
                   jax.ShapeDtypeStruct((NP, C), jnp.float32)),
    )(agg2, dis, b2.reshape(1, C))

    return logits[:N], soft[:N]
